# baseline (speedup 1.0000x reference)
import functools

import jax
import jax.numpy as jnp
from jax import lax
from jax.experimental import pallas as pl
from jax.experimental.pallas import tpu as pltpu
from jax.experimental.pallas import tpu_sc as plsc

NC = 2
NS = 16
NW = NC * NS

B_TOTAL = 16384 * 26
BPW = B_TOTAL // NW
GRP = 256
GROUPS = BPW // GRP
G = 4
CHUNKS = GROUPS // G
D = 32


def _sc_gather(x3d, table):
    mesh = plsc.VectorSubcoreMesh(core_axis_name="c", subcore_axis_name="s")

    @functools.partial(
        pl.kernel,
        mesh=mesh,
        out_type=jax.ShapeDtypeStruct((NW, GROUPS, GRP, D), jnp.float32),
        compiler_params=pltpu.CompilerParams(use_tc_tiling_on_sc=False),
        scratch_types=[
            pltpu.VMEM((GROUPS, GRP), jnp.int32),
            pltpu.VMEM((G, GRP, D), jnp.float32),
            pltpu.VMEM((G, GRP, D), jnp.float32),
            pltpu.SemaphoreType.DMA,
            pltpu.SemaphoreType.DMA,
        ],
    )
    def k(idx_hbm, table_hbm, out_hbm, idx_v, rows0, rows1, sem0, sem1):
        wid = lax.axis_index("s") * NC + lax.axis_index("c")
        pltpu.sync_copy(idx_hbm.at[wid], idx_v)

        rows = (rows0, rows1)
        sems = (sem0, sem1)

        def fire(j, buf):
            for g in range(G):
                pltpu.async_copy(
                    table_hbm.at[idx_v.at[j * G + g]], rows[buf].at[g], sems[buf]
                )

        def drain(j, buf):
            for g in range(G):
                pltpu.make_async_copy(
                    table_hbm.at[idx_v.at[j * G + g]], rows[buf].at[g], sems[buf]
                ).wait()
            pltpu.sync_copy(rows[buf], out_hbm.at[wid, pl.ds(j * G, G)])

        fire(0, 0)

        def body(i, carry):
            j = i * 2
            fire(j + 1, 1)
            drain(j, 0)
            fire(j + 2, 0)
            drain(j + 1, 1)
            return carry

        lax.fori_loop(0, (CHUNKS - 1) // 2, body, 0)
        drain(CHUNKS - 1, 0)

    return k(x3d, table)


def kernel(x, table):
    x3d = x.reshape(NW, GROUPS, GRP)
    out = _sc_gather(x3d, table)
    return out.reshape(16384, 26, D)


# speedup vs baseline: 1.0019x; 1.0019x over previous
import functools

import jax
import jax.numpy as jnp
from jax import lax
from jax.experimental import pallas as pl
from jax.experimental.pallas import tpu as pltpu
from jax.experimental.pallas import tpu_sc as plsc

NC = 2
NS = 16
NW = NC * NS

B = 16384
C = 26
D = 32
BPW = B // NW
GRP = 128
GPS = BPW // GRP


def _sc_gather(x_t, table):
    mesh = plsc.VectorSubcoreMesh(core_axis_name="c", subcore_axis_name="s")

    @functools.partial(
        pl.kernel,
        mesh=mesh,
        out_type=jax.ShapeDtypeStruct((B, C, D), jnp.float32),
        compiler_params=pltpu.CompilerParams(use_tc_tiling_on_sc=False),
        scratch_types=[
            pltpu.VMEM((C, BPW), jnp.int32),
            pltpu.VMEM((BPW, D), jnp.float32),
            pltpu.VMEM((BPW, D), jnp.float32),
            pltpu.SemaphoreType.DMA,
            pltpu.SemaphoreType.DMA,
            pltpu.SemaphoreType.DMA,
            pltpu.SemaphoreType.DMA,
        ],
    )
    def k(xt_hbm, table_hbm, out_hbm, idx_v, buf0, buf1, g0, g1, o0, o1):
        wid = lax.axis_index("s") * NC + lax.axis_index("c")
        base = wid * BPW
        pltpu.sync_copy(xt_hbm.at[:, pl.ds(base, BPW)], idx_v)

        bufs = (buf0, buf1)
        gsem = (g0, g1)
        osem = (o0, o1)

        def fire(c, p):
            for g in range(GPS):
                pltpu.async_copy(
                    table_hbm.at[idx_v.at[c, pl.ds(g * GRP, GRP)]],
                    bufs[p].at[pl.ds(g * GRP, GRP)],
                    gsem[p],
                )

        def drain(c, p):
            for g in range(GPS):
                pltpu.make_async_copy(
                    table_hbm.at[idx_v.at[c, pl.ds(g * GRP, GRP)]],
                    bufs[p].at[pl.ds(g * GRP, GRP)],
                    gsem[p],
                ).wait()

        def out_fire(c, p):
            pltpu.async_copy(bufs[p], out_hbm.at[pl.ds(base, BPW), c], osem[p])

        def out_wait(c, p):
            pltpu.make_async_copy(
                bufs[p], out_hbm.at[pl.ds(base, BPW), c], osem[p]
            ).wait()

        fire(0, 0)
        fire(1, 1)

        def body(i, carry):
            c = i * 2
            drain(c, 0)
            out_fire(c, 0)
            out_wait(c, 0)
            fire(c + 2, 0)
            drain(c + 1, 1)
            out_fire(c + 1, 1)
            out_wait(c + 1, 1)
            fire(c + 3, 1)
            return carry

        lax.fori_loop(0, (C - 2) // 2, body, 0)
        drain(C - 2, 0)
        out_fire(C - 2, 0)
        drain(C - 1, 1)
        out_fire(C - 1, 1)
        out_wait(C - 2, 0)
        out_wait(C - 1, 1)

    return k(x_t, table)


def kernel(x, table):
    return _sc_gather(x.T, table)
